# Initial kernel scaffold; baseline (speedup 1.0000x reference)
import jax
import jax.numpy as jnp
from jax.experimental import pallas as pl


def kernel(t, x, edge_index, c_mask, f_mask, wc_2, wf_2):
    raise NotImplementedError("write your pallas kernel here")


# speedup vs baseline: 69.8102x; 69.8102x over previous
import functools

import jax
import jax.numpy as jnp
from jax import lax
from jax.experimental import pallas as pl
from jax.experimental.pallas import tpu as pltpu
from jax.experimental.pallas import tpu_sc as plsc

N_NODES = 100000
N_EDGES = 1600000
NPAD = 102400
EPAD = 1638400
ROWS = EPAD // 128
NW = 32
RPW = ROWS // NW
RCH = 16
NCHUNK = RPW // RCH
SLICE = NPAD // 16

_mesh = plsc.VectorSubcoreMesh(core_axis_name="c", subcore_axis_name="s")


@functools.partial(
    pl.kernel,
    out_type=(
        jax.ShapeDtypeStruct((ROWS, 128), jnp.float32),
        jax.ShapeDtypeStruct((ROWS, 128), jnp.float32),
    ),
    mesh=_mesh,
    scratch_types=[
        pltpu.VMEM((NPAD,), jnp.float32),
        pltpu.VMEM((RCH, 128), jnp.int32),
        pltpu.VMEM((RCH, 128), jnp.int32),
        pltpu.VMEM((RCH, 128), jnp.float32),
        pltpu.VMEM((RCH, 128), jnp.float32),
    ],
    compiler_params=pltpu.CompilerParams(needs_layout_passes=False),
)
def _gather_call(x_hbm, src_hbm, dst_hbm, xj_hbm, xi_hbm,
                 table_v, src_v, dst_v, xj_v, xi_v):
    cid = lax.axis_index("c")
    sid = lax.axis_index("s")
    wid = cid * 16 + sid
    pltpu.sync_copy(x_hbm, table_v)

    def chunk(ci, _):
        rb = wid * RPW + ci * RCH
        pltpu.sync_copy(src_hbm.at[pl.ds(rb, RCH)], src_v)
        pltpu.sync_copy(dst_hbm.at[pl.ds(rb, RCH)], dst_v)

        def row(j, _):
            for c in range(8):
                sl = pl.ds(c * 16, 16)
                xj_v[j, sl] = plsc.load_gather(table_v, [src_v[j, sl]])
                xi_v[j, sl] = plsc.load_gather(table_v, [dst_v[j, sl]])
            return 0

        lax.fori_loop(0, RCH, row, 0)
        pltpu.sync_copy(xj_v, xj_hbm.at[pl.ds(rb, RCH)])
        pltpu.sync_copy(xi_v, xi_hbm.at[pl.ds(rb, RCH)])
        return 0

    lax.fori_loop(0, NCHUNK, chunk, 0)


@functools.partial(
    pl.kernel,
    out_type=jax.ShapeDtypeStruct((NW, SLICE), jnp.float32),
    mesh=_mesh,
    scratch_types=[
        pltpu.VMEM_SHARED((NPAD,), jnp.float32),
        pltpu.VMEM((RCH, 128), jnp.int32),
        pltpu.VMEM((RCH, 128), jnp.float32),
        pltpu.VMEM((SLICE,), jnp.float32),
    ],
    compiler_params=pltpu.CompilerParams(needs_layout_passes=False),
)
def _scatter_call(msg_hbm, dst_hbm, acc_hbm, shared_v, idx_v, msg_v, buf_v):
    cid = lax.axis_index("c")
    sid = lax.axis_index("s")
    wid = cid * 16 + sid

    def zfill(i, _):
        buf_v[pl.ds(i * 16, 16)] = jnp.zeros((16,), jnp.float32)
        return 0

    lax.fori_loop(0, SLICE // 16, zfill, 0)
    pltpu.sync_copy(buf_v, shared_v.at[pl.ds(sid * SLICE, SLICE)])
    plsc.subcore_barrier()

    def chunk(ci, _):
        rb = wid * RPW + ci * RCH
        pltpu.sync_copy(dst_hbm.at[pl.ds(rb, RCH)], idx_v)
        pltpu.sync_copy(msg_hbm.at[pl.ds(rb, RCH)], msg_v)
        for j in range(RCH):
            pltpu.sync_copy(msg_v.at[j], shared_v.at[idx_v.at[j]], add=True)
        return 0

    lax.fori_loop(0, NCHUNK, chunk, 0)
    plsc.subcore_barrier()
    pltpu.sync_copy(shared_v.at[pl.ds(sid * SLICE, SLICE)], acc_hbm.at[wid])


def _edge_body(c_ref, w2_ref, xj_ref, xi_ref, o_ref):
    a = xi_ref[...]
    b = xj_ref[...]
    d = b - a
    s = a + b

    def w(k):
        return c_ref[k, 0] * (w2_ref[k, 0] - w2_ref[k + 38, 0])

    def inv(v):
        return v / (v * v + 0.1)

    ia, ib, idf, iab = inv(a), inv(b), inv(d), inv(a * b)
    sind, cosd, tand = jnp.sin(d), jnp.cos(d), jnp.tanh(d)
    sins, coss, tans = jnp.sin(s), jnp.cos(s), jnp.tanh(s)
    acc = w(0) * a + w(1) * b + w(2) * d
    acc += w(3) * (a * a) + w(4) * (b * b) + w(5) * (d * d)
    acc += w(6) * ia + w(7) * ib + w(8) * idf + w(9) * iab
    acc += w(10) * (ia * ia) + w(11) * (ib * ib)
    acc += w(12) * (idf * idf) + w(13) * (iab * iab)
    acc += w(14) * (a * b) + w(15) * (a * a * b) + w(16) * (a * b * b)
    acc += w(17) * sind + w(18) * cosd + w(19) * tand
    acc += w(20) * sins + w(21) * coss
    acc += w(22) + w(23) * a + w(24) * b + w(25) * d
    acc += w(26) * jnp.sin(a) + w(27) * jnp.cos(a) + w(28) * jnp.tanh(a)
    acc += w(29) * jnp.sin(b) + w(30) * jnp.cos(b) + w(31) * jnp.tanh(b)
    acc += w(32) * sind + w(33) * cosd + w(34) * tand
    acc += w(35) * sins + w(36) * coss + w(37) * tans
    o_ref[...] = acc


_EBR = 1280
_edge_call = pl.pallas_call(
    _edge_body,
    grid=(ROWS // _EBR,),
    in_specs=[
        pl.BlockSpec((38, 1), lambda i: (0, 0)),
        pl.BlockSpec((76, 1), lambda i: (0, 0)),
        pl.BlockSpec((_EBR, 128), lambda i: (i, 0)),
        pl.BlockSpec((_EBR, 128), lambda i: (i, 0)),
    ],
    out_specs=pl.BlockSpec((_EBR, 128), lambda i: (i, 0)),
    out_shape=jax.ShapeDtypeStruct((ROWS, 128), jnp.float32),
)


def _combine_body(f_ref, w2_ref, acc_ref, x_ref, o_ref):
    x = x_ref[...]

    def w(k):
        return f_ref[k, 0] * (w2_ref[k, 0] - w2_ref[k + 12, 0])

    iv = x / (x * x + 0.1)
    g = w(0) + (w(1) + w(3)) * x + w(2) * jnp.sign(x)
    g += w(4) * (x * x) + w(5) * (x * x * x)
    g += w(6) * iv + w(7) * (iv * iv) + w(8) * (iv * iv * iv)
    g += w(9) * jnp.sin(x) + w(10) * jnp.cos(x) + w(11) * jnp.tanh(x)
    o_ref[...] = acc_ref[0] + acc_ref[1] + 0.1 * g


_CBR = 200
_combine_call = pl.pallas_call(
    _combine_body,
    grid=(NPAD // 128 // _CBR,),
    in_specs=[
        pl.BlockSpec((12, 1), lambda i: (0, 0)),
        pl.BlockSpec((24, 1), lambda i: (0, 0)),
        pl.BlockSpec((2, _CBR, 128), lambda i: (0, i, 0)),
        pl.BlockSpec((_CBR, 128), lambda i: (i, 0)),
    ],
    out_specs=pl.BlockSpec((_CBR, 128), lambda i: (i, 0)),
    out_shape=jax.ShapeDtypeStruct((NPAD // 128, 128), jnp.float32),
)


def kernel(t, x, edge_index, c_mask, f_mask, wc_2, wf_2):
    epad = EPAD - N_EDGES
    src1 = jnp.concatenate(
        [edge_index[0], jnp.zeros((epad,), jnp.int32)]).reshape(ROWS, 128)
    dst1 = jnp.concatenate(
        [edge_index[1], jnp.full((epad,), N_NODES, jnp.int32)]).reshape(ROWS, 128)
    x_pad = jnp.pad(x.reshape(-1), (0, NPAD - N_NODES))

    xj, xi = _gather_call(x_pad, src1, dst1)
    msg = _edge_call(c_mask, wc_2, xj, xi)
    acc = _scatter_call(msg, dst1)
    out = _combine_call(f_mask, wf_2, acc.reshape(2, NPAD // 128, 128),
                        x_pad.reshape(NPAD // 128, 128))
    return out.reshape(-1)[:N_NODES].reshape(N_NODES, 1)


# speedup vs baseline: 128.8331x; 1.8455x over previous
import functools

import jax
import jax.numpy as jnp
from jax import lax
from jax.experimental import pallas as pl
from jax.experimental.pallas import tpu as pltpu
from jax.experimental.pallas import tpu_sc as plsc

N_NODES = 100000
N_EDGES = 1600000
NPAD = 102400
EROWS = N_EDGES // 128
EPW = N_EDGES // 32
CH = 1280
NCH = 39
TAIL = EPW - NCH * CH
SLICE = NPAD // 16

_mesh = plsc.VectorSubcoreMesh(core_axis_name="c", subcore_axis_name="s")
_sc_params = pltpu.CompilerParams(needs_layout_passes=False)


@functools.partial(
    pl.kernel,
    out_type=(
        jax.ShapeDtypeStruct((N_EDGES,), jnp.float32),
        jax.ShapeDtypeStruct((N_EDGES,), jnp.float32),
    ),
    mesh=_mesh,
    scratch_types=[
        pltpu.VMEM((NPAD,), jnp.float32),
        pltpu.VMEM((CH,), jnp.int32),
        pltpu.VMEM((CH,), jnp.int32),
        pltpu.VMEM((CH,), jnp.int32),
        pltpu.VMEM((CH,), jnp.int32),
        pltpu.VMEM((CH,), jnp.float32),
        pltpu.VMEM((CH,), jnp.float32),
        pltpu.VMEM((CH,), jnp.float32),
        pltpu.VMEM((CH,), jnp.float32),
        pltpu.SemaphoreType.DMA,
        pltpu.SemaphoreType.DMA,
        pltpu.SemaphoreType.DMA,
        pltpu.SemaphoreType.DMA,
    ],
    compiler_params=_sc_params,
)
def _gather_call(x_hbm, ei_hbm, xj_hbm, xi_hbm,
                 table_v, srcA, dstA, srcB, dstB, xjA, xiA, xjB, xiB,
                 siA, siB, soA, soB):
    cid = lax.axis_index("c")
    sid = lax.axis_index("s")
    wbase = (cid * 16 + sid) * EPW
    pltpu.sync_copy(x_hbm, table_v)

    def start_in(eb, src_v, dst_v, sem, n=CH):
        pltpu.async_copy(ei_hbm.at[pl.ds(eb, n)], src_v.at[pl.ds(0, n)], sem)
        pltpu.async_copy(ei_hbm.at[pl.ds(N_EDGES + eb, n)], dst_v.at[pl.ds(0, n)], sem)

    def wait_in(eb, src_v, dst_v, sem, n=CH):
        pltpu.make_async_copy(ei_hbm.at[pl.ds(eb, n)], src_v.at[pl.ds(0, n)], sem).wait()
        pltpu.make_async_copy(ei_hbm.at[pl.ds(N_EDGES + eb, n)], dst_v.at[pl.ds(0, n)], sem).wait()

    def start_out(eb, xj_v, xi_v, sem, n=CH):
        pltpu.async_copy(xj_v.at[pl.ds(0, n)], xj_hbm.at[pl.ds(eb, n)], sem)
        pltpu.async_copy(xi_v.at[pl.ds(0, n)], xi_hbm.at[pl.ds(eb, n)], sem)

    def wait_out(eb, xj_v, xi_v, sem, n=CH):
        pltpu.make_async_copy(xj_v.at[pl.ds(0, n)], xj_hbm.at[pl.ds(eb, n)], sem).wait()
        pltpu.make_async_copy(xi_v.at[pl.ds(0, n)], xi_hbm.at[pl.ds(eb, n)], sem).wait()

    def gather(src_v, dst_v, xj_v, xi_v, ngrp=CH // 16):
        for g in range(ngrp):
            sl = pl.ds(g * 16, 16)
            xj_v[sl] = plsc.load_gather(table_v, [src_v[sl]])
            xi_v[sl] = plsc.load_gather(table_v, [dst_v[sl]])

    start_in(wbase, srcA, dstA, siA)
    start_in(wbase + CH, srcB, dstB, siB)

    def body(k, _):
        ebA = wbase + (2 * k) * CH
        ebB = wbase + (2 * k + 1) * CH
        wait_in(ebA, srcA, dstA, siA)

        @pl.when(k > 0)
        def _():
            wait_out(ebA, xjA, xiA, soA)

        gather(srcA, dstA, xjA, xiA)
        start_out(ebA, xjA, xiA, soA)
        start_in(ebA + 2 * CH, srcA, dstA, siA)

        wait_in(ebB, srcB, dstB, siB)

        @pl.when(k > 0)
        def _():
            wait_out(ebB, xjB, xiB, soB)

        gather(srcB, dstB, xjB, xiB)
        start_out(ebB, xjB, xiB, soB)

        @pl.when(k < 18)
        def _():
            start_in(ebB + 2 * CH, srcB, dstB, siB)

        return 0

    lax.fori_loop(0, 19, body, 0)

    eb38 = wbase + 38 * CH
    wait_in(eb38, srcA, dstA, siA)
    wait_out(eb38, xjA, xiA, soA)
    gather(srcA, dstA, xjA, xiA)
    start_out(eb38, xjA, xiA, soA)

    ebt = wbase + NCH * CH
    wait_out(ebt, xjB, xiB, soB)
    pltpu.sync_copy(ei_hbm.at[pl.ds(ebt, TAIL)], srcB.at[pl.ds(0, TAIL)])
    pltpu.sync_copy(ei_hbm.at[pl.ds(N_EDGES + ebt, TAIL)], dstB.at[pl.ds(0, TAIL)])
    gather(srcB, dstB, xjB, xiB, ngrp=TAIL // 16)
    pltpu.sync_copy(xjB.at[pl.ds(0, TAIL)], xj_hbm.at[pl.ds(ebt, TAIL)])
    pltpu.sync_copy(xiB.at[pl.ds(0, TAIL)], xi_hbm.at[pl.ds(ebt, TAIL)])
    wait_out(eb38, xjA, xiA, soA)


@functools.partial(
    pl.kernel,
    out_type=jax.ShapeDtypeStruct((32, SLICE), jnp.float32),
    mesh=_mesh,
    scratch_types=[
        pltpu.VMEM_SHARED((NPAD,), jnp.float32),
        pltpu.VMEM((CH,), jnp.int32),
        pltpu.VMEM((CH,), jnp.int32),
        pltpu.VMEM((CH,), jnp.float32),
        pltpu.VMEM((CH,), jnp.float32),
        pltpu.VMEM((SLICE,), jnp.float32),
        pltpu.SemaphoreType.DMA,
        pltpu.SemaphoreType.DMA,
    ],
    compiler_params=_sc_params,
)
def _scatter_call(msg_hbm, ei_hbm, acc_hbm,
                  shared_v, idxA, idxB, msgA, msgB, zbuf, siA, siB):
    cid = lax.axis_index("c")
    sid = lax.axis_index("s")
    wid = cid * 16 + sid
    wbase = wid * EPW

    def zfill(i, _):
        zbuf[pl.ds(i * 16, 16)] = jnp.zeros((16,), jnp.float32)
        return 0

    lax.fori_loop(0, SLICE // 16, zfill, 0)
    pltpu.sync_copy(zbuf, shared_v.at[pl.ds(sid * SLICE, SLICE)])
    plsc.subcore_barrier()

    def start_in(eb, idx_v, msg_v, sem, n=CH):
        pltpu.async_copy(ei_hbm.at[pl.ds(N_EDGES + eb, n)], idx_v.at[pl.ds(0, n)], sem)
        pltpu.async_copy(msg_hbm.at[pl.ds(eb, n)], msg_v.at[pl.ds(0, n)], sem)

    def wait_in(eb, idx_v, msg_v, sem, n=CH):
        pltpu.make_async_copy(ei_hbm.at[pl.ds(N_EDGES + eb, n)], idx_v.at[pl.ds(0, n)], sem).wait()
        pltpu.make_async_copy(msg_hbm.at[pl.ds(eb, n)], msg_v.at[pl.ds(0, n)], sem).wait()

    start_in(wbase, idxA, msgA, siA)
    start_in(wbase + CH, idxB, msgB, siB)

    def body(k, _):
        ebA = wbase + (2 * k) * CH
        ebB = wbase + (2 * k + 1) * CH
        wait_in(ebA, idxA, msgA, siA)
        pltpu.sync_copy(msgA, shared_v.at[idxA], add=True)
        start_in(ebA + 2 * CH, idxA, msgA, siA)
        wait_in(ebB, idxB, msgB, siB)
        pltpu.sync_copy(msgB, shared_v.at[idxB], add=True)

        @pl.when(k < 18)
        def _():
            start_in(ebB + 2 * CH, idxB, msgB, siB)

        return 0

    lax.fori_loop(0, 19, body, 0)
    eb38 = wbase + 38 * CH
    wait_in(eb38, idxA, msgA, siA)
    pltpu.sync_copy(msgA, shared_v.at[idxA], add=True)
    ebt = wbase + NCH * CH
    pltpu.sync_copy(ei_hbm.at[pl.ds(N_EDGES + ebt, TAIL)], idxB.at[pl.ds(0, TAIL)])
    pltpu.sync_copy(msg_hbm.at[pl.ds(ebt, TAIL)], msgB.at[pl.ds(0, TAIL)])
    pltpu.sync_copy(msgB.at[pl.ds(0, TAIL)],
                    shared_v.at[idxB.at[pl.ds(0, TAIL)]], add=True)

    plsc.subcore_barrier()
    pltpu.sync_copy(shared_v.at[pl.ds(sid * SLICE, SLICE)], acc_hbm.at[wid])


def _edge_body(c_ref, w2_ref, xj_ref, xi_ref, o_ref):
    a = xi_ref[...]
    b = xj_ref[...]
    d = b - a

    def w(k):
        return c_ref[k, 0] * (w2_ref[k, 0] - w2_ref[k + 38, 0])

    def inv(v):
        return v / (v * v + 0.1)

    ia, ib, idf, iab = inv(a), inv(b), inv(d), inv(a * b)
    sa, ca, ta = jnp.sin(a), jnp.cos(a), jnp.tanh(a)
    sb, cb, tb = jnp.sin(b), jnp.cos(b), jnp.tanh(b)
    tand = (tb - ta) / jnp.maximum(1.0 - ta * tb, 1e-20)
    tans = (ta + tb) / jnp.maximum(1.0 + ta * tb, 1e-20)
    p1, p2 = sa * cb, ca * sb
    p3, p4 = ca * cb, sa * sb
    A, B = w(17) + w(32), w(18) + w(33)
    C, D = w(20) + w(35), w(21) + w(36)
    acc = (C - A) * p1 + (A + C) * p2 + (B + D) * p3 + (B - D) * p4
    wd = w(2) + w(25)
    acc += (w(0) + w(23) - wd) * a + (w(1) + w(24) + wd) * b + w(22)
    w5 = w(5)
    ab = a * b
    acc += (w(3) + w5) * (a * a) + (w(4) + w5) * (b * b) + (w(14) - 2.0 * w5) * ab
    acc += w(15) * (ab * a) + w(16) * (ab * b)
    acc += w(6) * ia + w(7) * ib + w(8) * idf + w(9) * iab
    acc += w(10) * (ia * ia) + w(11) * (ib * ib)
    acc += w(12) * (idf * idf) + w(13) * (iab * iab)
    acc += (w(19) + w(34)) * tand + w(37) * tans
    acc += w(26) * sa + w(27) * ca + w(28) * ta
    acc += w(29) * sb + w(30) * cb + w(31) * tb
    o_ref[...] = acc


_EBR = 1000
_edge_call = pl.pallas_call(
    _edge_body,
    grid=(pl.cdiv(EROWS, _EBR),),
    in_specs=[
        pl.BlockSpec((38, 1), lambda i: (0, 0)),
        pl.BlockSpec((76, 1), lambda i: (0, 0)),
        pl.BlockSpec((_EBR, 128), lambda i: (i, 0)),
        pl.BlockSpec((_EBR, 128), lambda i: (i, 0)),
    ],
    out_specs=pl.BlockSpec((_EBR, 128), lambda i: (i, 0)),
    out_shape=jax.ShapeDtypeStruct((EROWS, 128), jnp.float32),
)


def _combine_body(f_ref, w2_ref, acc_ref, x_ref, o_ref):
    x = x_ref[...]

    def w(k):
        return f_ref[k, 0] * (w2_ref[k, 0] - w2_ref[k + 12, 0])

    iv = x / (x * x + 0.1)
    g = w(0) + (w(1) + w(3)) * x + w(2) * jnp.sign(x)
    g += w(4) * (x * x) + w(5) * (x * x * x)
    g += w(6) * iv + w(7) * (iv * iv) + w(8) * (iv * iv * iv)
    g += w(9) * jnp.sin(x) + w(10) * jnp.cos(x) + w(11) * jnp.tanh(x)
    o_ref[...] = acc_ref[0] + acc_ref[1] + 0.1 * g


_CBR = 200
_combine_call = pl.pallas_call(
    _combine_body,
    grid=(NPAD // 128 // _CBR,),
    in_specs=[
        pl.BlockSpec((12, 1), lambda i: (0, 0)),
        pl.BlockSpec((24, 1), lambda i: (0, 0)),
        pl.BlockSpec((2, _CBR, 128), lambda i: (0, i, 0)),
        pl.BlockSpec((_CBR, 128), lambda i: (i, 0)),
    ],
    out_specs=pl.BlockSpec((_CBR, 128), lambda i: (i, 0)),
    out_shape=jax.ShapeDtypeStruct((NPAD // 128, 128), jnp.float32),
)


def kernel(t, x, edge_index, c_mask, f_mask, wc_2, wf_2):
    x_pad = jnp.pad(x.reshape(-1), (0, NPAD - N_NODES))
    ei1 = edge_index.reshape(-1)
    xj, xi = _gather_call(x_pad, ei1)
    msg = _edge_call(c_mask, wc_2, xj.reshape(EROWS, 128), xi.reshape(EROWS, 128))
    acc = _scatter_call(msg.reshape(-1), ei1)
    out = _combine_call(f_mask, wf_2, acc.reshape(2, NPAD // 128, 128),
                        x_pad.reshape(NPAD // 128, 128))
    return out.reshape(-1)[:N_NODES].reshape(N_NODES, 1)


# speedup vs baseline: 146.8677x; 1.1400x over previous
import functools

import jax
import jax.numpy as jnp
from jax import lax
from jax.experimental import pallas as pl
from jax.experimental.pallas import tpu as pltpu
from jax.experimental.pallas import tpu_sc as plsc

N_NODES = 100000
N_EDGES = 1600000
NPAD = 102400
EROWS = N_EDGES // 128
CH = 1280
NCH = 39
EPW = NCH * CH
REM = N_EDGES - 32 * EPW
SLICE = NPAD // 16

_mesh = plsc.VectorSubcoreMesh(core_axis_name="c", subcore_axis_name="s")
_sc_params = pltpu.CompilerParams(needs_layout_passes=False)


@functools.partial(
    pl.kernel,
    out_type=(
        jax.ShapeDtypeStruct((N_EDGES,), jnp.float32),
        jax.ShapeDtypeStruct((N_EDGES,), jnp.float32),
        jax.ShapeDtypeStruct((N_EDGES,), jnp.int32),
    ),
    mesh=_mesh,
    scratch_types=[
        pltpu.VMEM((NPAD,), jnp.float32),
        pltpu.VMEM((2, CH), jnp.int32),
        pltpu.VMEM((2, CH), jnp.int32),
        pltpu.VMEM((CH,), jnp.float32),
        pltpu.VMEM((CH,), jnp.float32),
        pltpu.VMEM((CH,), jnp.float32),
        pltpu.VMEM((CH,), jnp.float32),
        pltpu.VMEM((CH,), jnp.int32),
        pltpu.VMEM((CH,), jnp.int32),
        pltpu.SemaphoreType.DMA,
        pltpu.SemaphoreType.DMA,
        pltpu.SemaphoreType.DMA,
        pltpu.SemaphoreType.DMA,
    ],
    compiler_params=_sc_params,
)
def _gather_call(x_hbm, ei_hbm, xj_hbm, xi_hbm, dst_hbm,
                 table_v, eiA, eiB, xjA, xiA, xjB, xiB, dstA, dstB,
                 siA, siB, soA, soB):
    cid = lax.axis_index("c")
    sid = lax.axis_index("s")
    wid = cid * 16 + sid
    wbase = wid * EPW
    pltpu.sync_copy(x_hbm, table_v)

    def start_in(eb, ei_v, sem, n=CH):
        pltpu.async_copy(ei_hbm.at[:, pl.ds(eb, n)], ei_v.at[:, pl.ds(0, n)], sem)

    def wait_in(eb, ei_v, sem, n=CH):
        pltpu.make_async_copy(ei_hbm.at[:, pl.ds(eb, n)], ei_v.at[:, pl.ds(0, n)], sem).wait()

    def start_out(eb, xj_v, xi_v, dst_v, sem, n=CH):
        pltpu.async_copy(xj_v.at[pl.ds(0, n)], xj_hbm.at[pl.ds(eb, n)], sem)
        pltpu.async_copy(xi_v.at[pl.ds(0, n)], xi_hbm.at[pl.ds(eb, n)], sem)
        pltpu.async_copy(dst_v.at[pl.ds(0, n)], dst_hbm.at[pl.ds(eb, n)], sem)

    def wait_out(eb, xj_v, xi_v, dst_v, sem, n=CH):
        pltpu.make_async_copy(xj_v.at[pl.ds(0, n)], xj_hbm.at[pl.ds(eb, n)], sem).wait()
        pltpu.make_async_copy(xi_v.at[pl.ds(0, n)], xi_hbm.at[pl.ds(eb, n)], sem).wait()
        pltpu.make_async_copy(dst_v.at[pl.ds(0, n)], dst_hbm.at[pl.ds(eb, n)], sem).wait()

    def gather(ei_v, xj_v, xi_v, dst_v, ngrp=CH // 16):
        for g in range(ngrp):
            sl = pl.ds(g * 16, 16)
            d_idx = ei_v[1, sl]
            xj_v[sl] = plsc.load_gather(table_v, [ei_v[0, sl]])
            xi_v[sl] = plsc.load_gather(table_v, [d_idx])
            dst_v[sl] = d_idx

    start_in(wbase, eiA, siA)
    start_in(wbase + CH, eiB, siB)

    def body(k, _):
        ebA = wbase + (2 * k) * CH
        ebB = wbase + (2 * k + 1) * CH
        wait_in(ebA, eiA, siA)

        @pl.when(k > 0)
        def _():
            wait_out(ebA, xjA, xiA, dstA, soA)

        gather(eiA, xjA, xiA, dstA)
        start_out(ebA, xjA, xiA, dstA, soA)
        start_in(ebA + 2 * CH, eiA, siA)

        wait_in(ebB, eiB, siB)

        @pl.when(k > 0)
        def _():
            wait_out(ebB, xjB, xiB, dstB, soB)

        gather(eiB, xjB, xiB, dstB)
        start_out(ebB, xjB, xiB, dstB, soB)

        @pl.when(k < 18)
        def _():
            start_in(ebB + 2 * CH, eiB, siB)

        return 0

    lax.fori_loop(0, 19, body, 0)

    eb38 = wbase + 38 * CH
    wait_in(eb38, eiA, siA)
    wait_out(eb38, xjA, xiA, dstA, soA)
    gather(eiA, xjA, xiA, dstA)
    start_out(eb38, xjA, xiA, dstA, soA)
    wait_out(wbase, xjB, xiB, dstB, soB)

    @pl.when(wid < REM // 128)
    def _():
        ebt = 32 * EPW + wid * 128
        pltpu.sync_copy(ei_hbm.at[:, pl.ds(ebt, 128)], eiB.at[:, pl.ds(0, 128)])
        gather(eiB, xjB, xiB, dstB, ngrp=128 // 16)
        pltpu.sync_copy(xjB.at[pl.ds(0, 128)], xj_hbm.at[pl.ds(ebt, 128)])
        pltpu.sync_copy(xiB.at[pl.ds(0, 128)], xi_hbm.at[pl.ds(ebt, 128)])
        pltpu.sync_copy(dstB.at[pl.ds(0, 128)], dst_hbm.at[pl.ds(ebt, 128)])

    wait_out(eb38, xjA, xiA, dstA, soA)


@functools.partial(
    pl.kernel,
    out_type=jax.ShapeDtypeStruct((32, SLICE), jnp.float32),
    mesh=_mesh,
    scratch_types=[
        pltpu.VMEM_SHARED((NPAD,), jnp.float32),
        pltpu.VMEM((CH,), jnp.int32),
        pltpu.VMEM((CH,), jnp.int32),
        pltpu.VMEM((CH,), jnp.float32),
        pltpu.VMEM((CH,), jnp.float32),
        pltpu.VMEM((SLICE,), jnp.float32),
        pltpu.SemaphoreType.DMA,
        pltpu.SemaphoreType.DMA,
    ],
    compiler_params=_sc_params,
)
def _scatter_call(msg_hbm, dst_hbm, acc_hbm,
                  shared_v, idxA, idxB, msgA, msgB, zbuf, siA, siB):
    cid = lax.axis_index("c")
    sid = lax.axis_index("s")
    wid = cid * 16 + sid
    wbase = wid * EPW

    def zfill(i, _):
        zbuf[pl.ds(i * 16, 16)] = jnp.zeros((16,), jnp.float32)
        return 0

    lax.fori_loop(0, SLICE // 16, zfill, 0)
    pltpu.sync_copy(zbuf, shared_v.at[pl.ds(sid * SLICE, SLICE)])
    plsc.subcore_barrier()

    def start_in(eb, idx_v, msg_v, sem, n=CH):
        pltpu.async_copy(dst_hbm.at[pl.ds(eb, n)], idx_v.at[pl.ds(0, n)], sem)
        pltpu.async_copy(msg_hbm.at[pl.ds(eb, n)], msg_v.at[pl.ds(0, n)], sem)

    def wait_in(eb, idx_v, msg_v, sem, n=CH):
        pltpu.make_async_copy(dst_hbm.at[pl.ds(eb, n)], idx_v.at[pl.ds(0, n)], sem).wait()
        pltpu.make_async_copy(msg_hbm.at[pl.ds(eb, n)], msg_v.at[pl.ds(0, n)], sem).wait()

    start_in(wbase, idxA, msgA, siA)
    start_in(wbase + CH, idxB, msgB, siB)

    def body(k, _):
        ebA = wbase + (2 * k) * CH
        ebB = wbase + (2 * k + 1) * CH
        wait_in(ebA, idxA, msgA, siA)
        pltpu.sync_copy(msgA, shared_v.at[idxA], add=True)
        start_in(ebA + 2 * CH, idxA, msgA, siA)
        wait_in(ebB, idxB, msgB, siB)
        pltpu.sync_copy(msgB, shared_v.at[idxB], add=True)

        @pl.when(k < 18)
        def _():
            start_in(ebB + 2 * CH, idxB, msgB, siB)

        return 0

    lax.fori_loop(0, 19, body, 0)
    eb38 = wbase + 38 * CH
    wait_in(eb38, idxA, msgA, siA)
    pltpu.sync_copy(msgA, shared_v.at[idxA], add=True)

    @pl.when(wid < REM // 128)
    def _():
        ebt = 32 * EPW + wid * 128
        pltpu.sync_copy(dst_hbm.at[pl.ds(ebt, 128)], idxB.at[pl.ds(0, 128)])
        pltpu.sync_copy(msg_hbm.at[pl.ds(ebt, 128)], msgB.at[pl.ds(0, 128)])
        pltpu.sync_copy(msgB.at[pl.ds(0, 128)],
                        shared_v.at[idxB.at[pl.ds(0, 128)]], add=True)

    plsc.subcore_barrier()
    pltpu.sync_copy(shared_v.at[pl.ds(sid * SLICE, SLICE)], acc_hbm.at[wid])


def _edge_body(c_ref, w2_ref, xj_ref, xi_ref, o_ref):
    a = xi_ref[...]
    b = xj_ref[...]
    d = b - a

    def w(k):
        return c_ref[k, 0] * (w2_ref[k, 0] - w2_ref[k + 38, 0])

    def inv(v):
        return v / (v * v + 0.1)

    ia, ib, idf, iab = inv(a), inv(b), inv(d), inv(a * b)
    sa, ca, ta = jnp.sin(a), jnp.cos(a), jnp.tanh(a)
    sb, cb, tb = jnp.sin(b), jnp.cos(b), jnp.tanh(b)
    tand = (tb - ta) / jnp.maximum(1.0 - ta * tb, 1e-20)
    tans = (ta + tb) / jnp.maximum(1.0 + ta * tb, 1e-20)
    p1, p2 = sa * cb, ca * sb
    p3, p4 = ca * cb, sa * sb
    A, B = w(17) + w(32), w(18) + w(33)
    C, D = w(20) + w(35), w(21) + w(36)
    acc = (C - A) * p1 + (A + C) * p2 + (B + D) * p3 + (B - D) * p4
    wd = w(2) + w(25)
    acc += (w(0) + w(23) - wd) * a + (w(1) + w(24) + wd) * b + w(22)
    w5 = w(5)
    ab = a * b
    acc += (w(3) + w5) * (a * a) + (w(4) + w5) * (b * b) + (w(14) - 2.0 * w5) * ab
    acc += w(15) * (ab * a) + w(16) * (ab * b)
    acc += w(6) * ia + w(7) * ib + w(8) * idf + w(9) * iab
    acc += w(10) * (ia * ia) + w(11) * (ib * ib)
    acc += w(12) * (idf * idf) + w(13) * (iab * iab)
    acc += (w(19) + w(34)) * tand + w(37) * tans
    acc += w(26) * sa + w(27) * ca + w(28) * ta
    acc += w(29) * sb + w(30) * cb + w(31) * tb
    o_ref[...] = acc


_EBR = 1000
_edge_call = pl.pallas_call(
    _edge_body,
    grid=(pl.cdiv(EROWS, _EBR),),
    in_specs=[
        pl.BlockSpec((38, 1), lambda i: (0, 0)),
        pl.BlockSpec((76, 1), lambda i: (0, 0)),
        pl.BlockSpec((_EBR, 128), lambda i: (i, 0)),
        pl.BlockSpec((_EBR, 128), lambda i: (i, 0)),
    ],
    out_specs=pl.BlockSpec((_EBR, 128), lambda i: (i, 0)),
    out_shape=jax.ShapeDtypeStruct((EROWS, 128), jnp.float32),
)


def _combine_body(f_ref, w2_ref, acc_ref, x_ref, o_ref):
    x = x_ref[...]

    def w(k):
        return f_ref[k, 0] * (w2_ref[k, 0] - w2_ref[k + 12, 0])

    iv = x / (x * x + 0.1)
    g = w(0) + (w(1) + w(3)) * x + w(2) * jnp.sign(x)
    g += w(4) * (x * x) + w(5) * (x * x * x)
    g += w(6) * iv + w(7) * (iv * iv) + w(8) * (iv * iv * iv)
    g += w(9) * jnp.sin(x) + w(10) * jnp.cos(x) + w(11) * jnp.tanh(x)
    o_ref[...] = acc_ref[0] + acc_ref[1] + 0.1 * g


_CBR = 200
_combine_call = pl.pallas_call(
    _combine_body,
    grid=(NPAD // 128 // _CBR,),
    in_specs=[
        pl.BlockSpec((12, 1), lambda i: (0, 0)),
        pl.BlockSpec((24, 1), lambda i: (0, 0)),
        pl.BlockSpec((2, _CBR, 128), lambda i: (0, i, 0)),
        pl.BlockSpec((_CBR, 128), lambda i: (i, 0)),
    ],
    out_specs=pl.BlockSpec((_CBR, 128), lambda i: (i, 0)),
    out_shape=jax.ShapeDtypeStruct((NPAD // 128, 128), jnp.float32),
)


def kernel(t, x, edge_index, c_mask, f_mask, wc_2, wf_2):
    x_pad = jnp.pad(x.reshape(-1), (0, NPAD - N_NODES))
    xj, xi, dst1 = _gather_call(x_pad, edge_index)
    msg = _edge_call(c_mask, wc_2, xj.reshape(EROWS, 128), xi.reshape(EROWS, 128))
    acc = _scatter_call(msg.reshape(-1), dst1)
    out = _combine_call(f_mask, wf_2, acc.reshape(2, NPAD // 128, 128),
                        x_pad.reshape(NPAD // 128, 128))
    return out.reshape(-1)[:N_NODES].reshape(N_NODES, 1)


# speedup vs baseline: 158.4349x; 1.0788x over previous
import functools

import jax
import jax.numpy as jnp
from jax import lax
from jax.experimental import pallas as pl
from jax.experimental.pallas import tpu as pltpu
from jax.experimental.pallas import tpu_sc as plsc

N_NODES = 100000
N_EDGES = 1600000
NPAD = 102400
EROWS = N_EDGES // 128
CH = 1280
NCH = 39
EPW = NCH * CH
REM = N_EDGES - 32 * EPW
SLICE = NPAD // 16

_mesh = plsc.VectorSubcoreMesh(core_axis_name="c", subcore_axis_name="s")
_sc_params = pltpu.CompilerParams(needs_layout_passes=False)


@functools.partial(
    pl.kernel,
    out_type=(
        jax.ShapeDtypeStruct((N_EDGES,), jnp.float32),
        jax.ShapeDtypeStruct((N_EDGES,), jnp.float32),
        jax.ShapeDtypeStruct((N_EDGES,), jnp.int32),
    ),
    mesh=_mesh,
    scratch_types=[
        pltpu.VMEM((NPAD,), jnp.float32),
        pltpu.VMEM((2, CH), jnp.int32),
        pltpu.VMEM((2, CH), jnp.int32),
        pltpu.VMEM((CH,), jnp.float32),
        pltpu.VMEM((CH,), jnp.float32),
        pltpu.VMEM((CH,), jnp.float32),
        pltpu.VMEM((CH,), jnp.float32),
        pltpu.VMEM((CH,), jnp.int32),
        pltpu.VMEM((CH,), jnp.int32),
        pltpu.SemaphoreType.DMA,
        pltpu.SemaphoreType.DMA,
        pltpu.SemaphoreType.DMA,
        pltpu.SemaphoreType.DMA,
    ],
    compiler_params=_sc_params,
)
def _gather_call(x_hbm, ei_hbm, xj_hbm, xi_hbm, dst_hbm,
                 table_v, eiA, eiB, xjA, xiA, xjB, xiB, dstA, dstB,
                 siA, siB, soA, soB):
    cid = lax.axis_index("c")
    sid = lax.axis_index("s")
    wid = cid * 16 + sid
    wbase = wid * EPW
    pltpu.sync_copy(x_hbm, table_v)

    def start_in(eb, ei_v, sem, n=CH):
        pltpu.async_copy(ei_hbm.at[:, pl.ds(eb, n)], ei_v.at[:, pl.ds(0, n)], sem)

    def wait_in(eb, ei_v, sem, n=CH):
        pltpu.make_async_copy(ei_hbm.at[:, pl.ds(eb, n)], ei_v.at[:, pl.ds(0, n)], sem).wait()

    def start_out(eb, xj_v, xi_v, dst_v, sem, n=CH):
        pltpu.async_copy(xj_v.at[pl.ds(0, n)], xj_hbm.at[pl.ds(eb, n)], sem)
        pltpu.async_copy(xi_v.at[pl.ds(0, n)], xi_hbm.at[pl.ds(eb, n)], sem)
        pltpu.async_copy(dst_v.at[pl.ds(0, n)], dst_hbm.at[pl.ds(eb, n)], sem)

    def wait_out(eb, xj_v, xi_v, dst_v, sem, n=CH):
        pltpu.make_async_copy(xj_v.at[pl.ds(0, n)], xj_hbm.at[pl.ds(eb, n)], sem).wait()
        pltpu.make_async_copy(xi_v.at[pl.ds(0, n)], xi_hbm.at[pl.ds(eb, n)], sem).wait()
        pltpu.make_async_copy(dst_v.at[pl.ds(0, n)], dst_hbm.at[pl.ds(eb, n)], sem).wait()

    def gather(ei_v, xj_v, xi_v, dst_v, ngrp=CH // 16):
        for g in range(ngrp):
            sl = pl.ds(g * 16, 16)
            d_idx = ei_v[1, sl]
            xj_v[sl] = plsc.load_gather(table_v, [ei_v[0, sl]])
            xi_v[sl] = plsc.load_gather(table_v, [d_idx])
            dst_v[sl] = d_idx

    start_in(wbase, eiA, siA)
    start_in(wbase + CH, eiB, siB)

    def body(k, _):
        ebA = wbase + (2 * k) * CH
        ebB = wbase + (2 * k + 1) * CH
        wait_in(ebA, eiA, siA)

        @pl.when(k > 0)
        def _():
            wait_out(ebA, xjA, xiA, dstA, soA)

        gather(eiA, xjA, xiA, dstA)
        start_out(ebA, xjA, xiA, dstA, soA)
        start_in(ebA + 2 * CH, eiA, siA)

        wait_in(ebB, eiB, siB)

        @pl.when(k > 0)
        def _():
            wait_out(ebB, xjB, xiB, dstB, soB)

        gather(eiB, xjB, xiB, dstB)
        start_out(ebB, xjB, xiB, dstB, soB)

        @pl.when(k < 18)
        def _():
            start_in(ebB + 2 * CH, eiB, siB)

        return 0

    lax.fori_loop(0, 19, body, 0)

    eb38 = wbase + 38 * CH
    wait_in(eb38, eiA, siA)
    wait_out(eb38, xjA, xiA, dstA, soA)
    gather(eiA, xjA, xiA, dstA)
    start_out(eb38, xjA, xiA, dstA, soA)
    wait_out(wbase, xjB, xiB, dstB, soB)

    @pl.when(wid < REM // 128)
    def _():
        ebt = 32 * EPW + wid * 128
        pltpu.sync_copy(ei_hbm.at[:, pl.ds(ebt, 128)], eiB.at[:, pl.ds(0, 128)])
        gather(eiB, xjB, xiB, dstB, ngrp=128 // 16)
        pltpu.sync_copy(xjB.at[pl.ds(0, 128)], xj_hbm.at[pl.ds(ebt, 128)])
        pltpu.sync_copy(xiB.at[pl.ds(0, 128)], xi_hbm.at[pl.ds(ebt, 128)])
        pltpu.sync_copy(dstB.at[pl.ds(0, 128)], dst_hbm.at[pl.ds(ebt, 128)])

    wait_out(eb38, xjA, xiA, dstA, soA)


@functools.partial(
    pl.kernel,
    out_type=jax.ShapeDtypeStruct((32, SLICE), jnp.float32),
    mesh=_mesh,
    scratch_types=[
        pltpu.VMEM_SHARED((NPAD,), jnp.float32),
        pltpu.VMEM((CH,), jnp.int32),
        pltpu.VMEM((CH,), jnp.int32),
        pltpu.VMEM((CH,), jnp.float32),
        pltpu.VMEM((CH,), jnp.float32),
        pltpu.VMEM((SLICE,), jnp.float32),
        pltpu.SemaphoreType.DMA,
        pltpu.SemaphoreType.DMA,
    ],
    compiler_params=_sc_params,
)
def _scatter_call(msg_hbm, dst_hbm, acc_hbm,
                  shared_v, idxA, idxB, msgA, msgB, zbuf, siA, siB):
    cid = lax.axis_index("c")
    sid = lax.axis_index("s")
    wid = cid * 16 + sid
    wbase = wid * EPW

    def zfill(i, _):
        zbuf[pl.ds(i * 16, 16)] = jnp.zeros((16,), jnp.float32)
        return 0

    lax.fori_loop(0, SLICE // 16, zfill, 0)
    pltpu.sync_copy(zbuf, shared_v.at[pl.ds(sid * SLICE, SLICE)])
    plsc.subcore_barrier()

    def start_in(eb, idx_v, msg_v, sem, n=CH):
        pltpu.async_copy(dst_hbm.at[pl.ds(eb, n)], idx_v.at[pl.ds(0, n)], sem)
        pltpu.async_copy(msg_hbm.at[pl.ds(eb, n)], msg_v.at[pl.ds(0, n)], sem)

    def wait_in(eb, idx_v, msg_v, sem, n=CH):
        pltpu.make_async_copy(dst_hbm.at[pl.ds(eb, n)], idx_v.at[pl.ds(0, n)], sem).wait()
        pltpu.make_async_copy(msg_hbm.at[pl.ds(eb, n)], msg_v.at[pl.ds(0, n)], sem).wait()

    start_in(wbase, idxA, msgA, siA)
    start_in(wbase + CH, idxB, msgB, siB)

    def body(k, _):
        ebA = wbase + (2 * k) * CH
        ebB = wbase + (2 * k + 1) * CH
        wait_in(ebA, idxA, msgA, siA)
        pltpu.sync_copy(msgA, shared_v.at[idxA], add=True)
        start_in(ebA + 2 * CH, idxA, msgA, siA)
        wait_in(ebB, idxB, msgB, siB)
        pltpu.sync_copy(msgB, shared_v.at[idxB], add=True)

        @pl.when(k < 18)
        def _():
            start_in(ebB + 2 * CH, idxB, msgB, siB)

        return 0

    lax.fori_loop(0, 19, body, 0)
    eb38 = wbase + 38 * CH
    wait_in(eb38, idxA, msgA, siA)
    pltpu.sync_copy(msgA, shared_v.at[idxA], add=True)

    @pl.when(wid < REM // 128)
    def _():
        ebt = 32 * EPW + wid * 128
        pltpu.sync_copy(dst_hbm.at[pl.ds(ebt, 128)], idxB.at[pl.ds(0, 128)])
        pltpu.sync_copy(msg_hbm.at[pl.ds(ebt, 128)], msgB.at[pl.ds(0, 128)])
        pltpu.sync_copy(msgB.at[pl.ds(0, 128)],
                        shared_v.at[idxB.at[pl.ds(0, 128)]], add=True)

    plsc.subcore_barrier()
    pltpu.sync_copy(shared_v.at[pl.ds(sid * SLICE, SLICE)], acc_hbm.at[wid])


def _sincos(v):
    t = v * 0.6366197723675814
    qf = (t + 12582912.0) - 12582912.0
    q = qf.astype(jnp.int32)
    r = v - qf * jnp.float32(1.5707963705062866)
    r = r + qf * jnp.float32(4.371138828673793e-8)
    s2 = r * r
    sinp = r + r * s2 * (-1.6666654611e-1 + s2 * (8.3321608736e-3 + s2 * (-1.9515295891e-4)))
    cosp = 1.0 + s2 * (-0.5 + s2 * (4.166664568298827e-2 + s2 * (-1.388731625493765e-3 + s2 * 2.443315711809948e-5)))
    swap = (q & 1) != 0
    sbase = jnp.where(swap, cosp, sinp)
    cbase = jnp.where(swap, sinp, cosp)
    s = jnp.where((q & 2) != 0, -sbase, sbase)
    c = jnp.where(((q + 1) & 2) != 0, -cbase, cbase)
    return s, c


def _edge_body(c_ref, w2_ref, xj_ref, xi_ref, o_ref):
    a = xi_ref[...]
    b = xj_ref[...]
    d = b - a

    def w(k):
        return c_ref[k, 0] * (w2_ref[k, 0] - w2_ref[k + 38, 0])

    def inv(v):
        return v / (v * v + 0.1)

    ia, ib, idf, iab = inv(a), inv(b), inv(d), inv(a * b)
    sa, ca = _sincos(a)
    sb, cb = _sincos(b)
    ta, tb = jnp.tanh(a), jnp.tanh(b)
    tand = (tb - ta) / jnp.maximum(1.0 - ta * tb, 1e-20)
    tans = (ta + tb) / jnp.maximum(1.0 + ta * tb, 1e-20)
    p1, p2 = sa * cb, ca * sb
    p3, p4 = ca * cb, sa * sb
    A, B = w(17) + w(32), w(18) + w(33)
    C, D = w(20) + w(35), w(21) + w(36)
    acc = (C - A) * p1 + (A + C) * p2 + (B + D) * p3 + (B - D) * p4
    wd = w(2) + w(25)
    acc += (w(0) + w(23) - wd) * a + (w(1) + w(24) + wd) * b + w(22)
    w5 = w(5)
    ab = a * b
    acc += (w(3) + w5) * (a * a) + (w(4) + w5) * (b * b) + (w(14) - 2.0 * w5) * ab
    acc += w(15) * (ab * a) + w(16) * (ab * b)
    acc += w(6) * ia + w(7) * ib + w(8) * idf + w(9) * iab
    acc += w(10) * (ia * ia) + w(11) * (ib * ib)
    acc += w(12) * (idf * idf) + w(13) * (iab * iab)
    acc += (w(19) + w(34)) * tand + w(37) * tans
    acc += w(26) * sa + w(27) * ca + w(28) * ta
    acc += w(29) * sb + w(30) * cb + w(31) * tb
    o_ref[...] = acc


_EBR = 1000
_edge_call = pl.pallas_call(
    _edge_body,
    grid=(pl.cdiv(EROWS, _EBR),),
    in_specs=[
        pl.BlockSpec((38, 1), lambda i: (0, 0)),
        pl.BlockSpec((76, 1), lambda i: (0, 0)),
        pl.BlockSpec((_EBR, 128), lambda i: (i, 0)),
        pl.BlockSpec((_EBR, 128), lambda i: (i, 0)),
    ],
    out_specs=pl.BlockSpec((_EBR, 128), lambda i: (i, 0)),
    out_shape=jax.ShapeDtypeStruct((EROWS, 128), jnp.float32),
)


def _combine_body(f_ref, w2_ref, acc_ref, x_ref, o_ref):
    x = x_ref[...]

    def w(k):
        return f_ref[k, 0] * (w2_ref[k, 0] - w2_ref[k + 12, 0])

    iv = x / (x * x + 0.1)
    g = w(0) + (w(1) + w(3)) * x + w(2) * jnp.sign(x)
    g += w(4) * (x * x) + w(5) * (x * x * x)
    g += w(6) * iv + w(7) * (iv * iv) + w(8) * (iv * iv * iv)
    g += w(9) * jnp.sin(x) + w(10) * jnp.cos(x) + w(11) * jnp.tanh(x)
    o_ref[...] = acc_ref[0] + acc_ref[1] + 0.1 * g


_CBR = 200
_combine_call = pl.pallas_call(
    _combine_body,
    grid=(NPAD // 128 // _CBR,),
    in_specs=[
        pl.BlockSpec((12, 1), lambda i: (0, 0)),
        pl.BlockSpec((24, 1), lambda i: (0, 0)),
        pl.BlockSpec((2, _CBR, 128), lambda i: (0, i, 0)),
        pl.BlockSpec((_CBR, 128), lambda i: (i, 0)),
    ],
    out_specs=pl.BlockSpec((_CBR, 128), lambda i: (i, 0)),
    out_shape=jax.ShapeDtypeStruct((NPAD // 128, 128), jnp.float32),
)


def kernel(t, x, edge_index, c_mask, f_mask, wc_2, wf_2):
    x_pad = jnp.pad(x.reshape(-1), (0, NPAD - N_NODES))
    xj, xi, dst1 = _gather_call(x_pad, edge_index)
    msg = _edge_call(c_mask, wc_2, xj.reshape(EROWS, 128), xi.reshape(EROWS, 128))
    acc = _scatter_call(msg.reshape(-1), dst1)
    out = _combine_call(f_mask, wf_2, acc.reshape(2, NPAD // 128, 128),
                        x_pad.reshape(NPAD // 128, 128))
    return out.reshape(-1)[:N_NODES].reshape(N_NODES, 1)


# speedup vs baseline: 167.0953x; 1.0547x over previous
import functools

import jax
import jax.numpy as jnp
from jax import lax
from jax.experimental import pallas as pl
from jax.experimental.pallas import tpu as pltpu
from jax.experimental.pallas import tpu_sc as plsc

N_NODES = 100000
N_EDGES = 1600000
NPAD = 102400
EH = N_EDGES // 2
HROWS = EH // 128
CH = 640
NCH = 39
EPW = NCH * CH
REM = EH - 32 * EPW
NMINI = REM // 128
SLICE = NPAD // 16

_mesh = plsc.VectorSubcoreMesh(core_axis_name="c", subcore_axis_name="s")
_sc_params = pltpu.CompilerParams(needs_layout_passes=False)


def _make_gather(e0):
    @functools.partial(
        pl.kernel,
        out_type=(
            jax.ShapeDtypeStruct((EH,), jnp.float32),
            jax.ShapeDtypeStruct((EH,), jnp.float32),
            jax.ShapeDtypeStruct((EH,), jnp.int32),
        ),
        mesh=_mesh,
        scratch_types=[
            pltpu.VMEM((NPAD,), jnp.float32),
            pltpu.VMEM((2, CH), jnp.int32),
            pltpu.VMEM((2, CH), jnp.int32),
            pltpu.VMEM((CH,), jnp.float32),
            pltpu.VMEM((CH,), jnp.float32),
            pltpu.VMEM((CH,), jnp.float32),
            pltpu.VMEM((CH,), jnp.float32),
            pltpu.VMEM((CH,), jnp.int32),
            pltpu.VMEM((CH,), jnp.int32),
            pltpu.SemaphoreType.DMA,
            pltpu.SemaphoreType.DMA,
            pltpu.SemaphoreType.DMA,
            pltpu.SemaphoreType.DMA,
        ],
        compiler_params=_sc_params,
    )
    def gather_kernel(x_hbm, ei_hbm, xj_hbm, xi_hbm, dst_hbm,
                      table_v, eiA, eiB, xjA, xiA, xjB, xiB, dstA, dstB,
                      siA, siB, soA, soB):
        cid = lax.axis_index("c")
        sid = lax.axis_index("s")
        wid = cid * 16 + sid
        wbase = e0 + wid * EPW

        def start_in(eb, ei_v, sem, n=CH):
            pltpu.async_copy(ei_hbm.at[:, pl.ds(eb, n)], ei_v.at[:, pl.ds(0, n)], sem)

        def wait_in(eb, ei_v, sem, n=CH):
            pltpu.make_async_copy(ei_hbm.at[:, pl.ds(eb, n)], ei_v.at[:, pl.ds(0, n)], sem).wait()

        def start_out(eb, xj_v, xi_v, dst_v, sem, n=CH):
            pltpu.async_copy(xj_v.at[pl.ds(0, n)], xj_hbm.at[pl.ds(eb - e0, n)], sem)
            pltpu.async_copy(xi_v.at[pl.ds(0, n)], xi_hbm.at[pl.ds(eb - e0, n)], sem)
            pltpu.async_copy(dst_v.at[pl.ds(0, n)], dst_hbm.at[pl.ds(eb - e0, n)], sem)

        def wait_out(eb, xj_v, xi_v, dst_v, sem, n=CH):
            pltpu.make_async_copy(xj_v.at[pl.ds(0, n)], xj_hbm.at[pl.ds(eb - e0, n)], sem).wait()
            pltpu.make_async_copy(xi_v.at[pl.ds(0, n)], xi_hbm.at[pl.ds(eb - e0, n)], sem).wait()
            pltpu.make_async_copy(dst_v.at[pl.ds(0, n)], dst_hbm.at[pl.ds(eb - e0, n)], sem).wait()

        def gather(ei_v, xj_v, xi_v, dst_v, ngrp=CH // 16):
            for g in range(ngrp):
                sl = pl.ds(g * 16, 16)
                d_idx = ei_v[1, sl]
                xj_v[sl] = plsc.load_gather(table_v, [ei_v[0, sl]])
                xi_v[sl] = plsc.load_gather(table_v, [d_idx])
                dst_v[sl] = d_idx

        start_in(wbase, eiA, siA)
        start_in(wbase + CH, eiB, siB)
        pltpu.sync_copy(x_hbm, table_v)

        def body(k, _):
            ebA = wbase + (2 * k) * CH
            ebB = wbase + (2 * k + 1) * CH
            wait_in(ebA, eiA, siA)

            @pl.when(k > 0)
            def _():
                wait_out(ebA, xjA, xiA, dstA, soA)

            gather(eiA, xjA, xiA, dstA)
            start_out(ebA, xjA, xiA, dstA, soA)
            start_in(ebA + 2 * CH, eiA, siA)

            wait_in(ebB, eiB, siB)

            @pl.when(k > 0)
            def _():
                wait_out(ebB, xjB, xiB, dstB, soB)

            gather(eiB, xjB, xiB, dstB)
            start_out(ebB, xjB, xiB, dstB, soB)

            @pl.when(k < 18)
            def _():
                start_in(ebB + 2 * CH, eiB, siB)

            return 0

        lax.fori_loop(0, 19, body, 0)

        eb38 = wbase + 38 * CH
        wait_in(eb38, eiA, siA)
        wait_out(eb38, xjA, xiA, dstA, soA)
        gather(eiA, xjA, xiA, dstA)
        start_out(eb38, xjA, xiA, dstA, soA)
        wait_out(wbase, xjB, xiB, dstB, soB)

        @pl.when(wid < NMINI)
        def _():
            ebt = e0 + 32 * EPW + wid * 128
            pltpu.sync_copy(ei_hbm.at[:, pl.ds(ebt, 128)], eiB.at[:, pl.ds(0, 128)])
            gather(eiB, xjB, xiB, dstB, ngrp=128 // 16)
            pltpu.sync_copy(xjB.at[pl.ds(0, 128)], xj_hbm.at[pl.ds(ebt - e0, 128)])
            pltpu.sync_copy(xiB.at[pl.ds(0, 128)], xi_hbm.at[pl.ds(ebt - e0, 128)])
            pltpu.sync_copy(dstB.at[pl.ds(0, 128)], dst_hbm.at[pl.ds(ebt - e0, 128)])

        wait_out(eb38, xjA, xiA, dstA, soA)

    return gather_kernel


@functools.partial(
    pl.kernel,
    out_type=jax.ShapeDtypeStruct((32, SLICE), jnp.float32),
    mesh=_mesh,
    scratch_types=[
        pltpu.VMEM_SHARED((NPAD,), jnp.float32),
        pltpu.VMEM((CH,), jnp.int32),
        pltpu.VMEM((CH,), jnp.int32),
        pltpu.VMEM((CH,), jnp.float32),
        pltpu.VMEM((CH,), jnp.float32),
        pltpu.VMEM((SLICE,), jnp.float32),
        pltpu.SemaphoreType.DMA,
        pltpu.SemaphoreType.DMA,
    ],
    compiler_params=_sc_params,
)
def _scatter_call(msg_hbm, dst_hbm, acc_hbm,
                  shared_v, idxA, idxB, msgA, msgB, zbuf, siA, siB):
    cid = lax.axis_index("c")
    sid = lax.axis_index("s")
    wid = cid * 16 + sid
    wbase = wid * EPW

    def start_in(eb, idx_v, msg_v, sem, n=CH):
        pltpu.async_copy(dst_hbm.at[pl.ds(eb, n)], idx_v.at[pl.ds(0, n)], sem)
        pltpu.async_copy(msg_hbm.at[pl.ds(eb, n)], msg_v.at[pl.ds(0, n)], sem)

    def wait_in(eb, idx_v, msg_v, sem, n=CH):
        pltpu.make_async_copy(dst_hbm.at[pl.ds(eb, n)], idx_v.at[pl.ds(0, n)], sem).wait()
        pltpu.make_async_copy(msg_hbm.at[pl.ds(eb, n)], msg_v.at[pl.ds(0, n)], sem).wait()

    start_in(wbase, idxA, msgA, siA)
    start_in(wbase + CH, idxB, msgB, siB)

    def zfill(i, _):
        zbuf[pl.ds(i * 16, 16)] = jnp.zeros((16,), jnp.float32)
        return 0

    lax.fori_loop(0, SLICE // 16, zfill, 0)
    pltpu.sync_copy(zbuf, shared_v.at[pl.ds(sid * SLICE, SLICE)])
    plsc.subcore_barrier()

    def body(k, _):
        ebA = wbase + (2 * k) * CH
        ebB = wbase + (2 * k + 1) * CH
        wait_in(ebA, idxA, msgA, siA)
        pltpu.sync_copy(msgA, shared_v.at[idxA], add=True)
        start_in(ebA + 2 * CH, idxA, msgA, siA)
        wait_in(ebB, idxB, msgB, siB)
        pltpu.sync_copy(msgB, shared_v.at[idxB], add=True)

        @pl.when(k < 18)
        def _():
            start_in(ebB + 2 * CH, idxB, msgB, siB)

        return 0

    lax.fori_loop(0, 19, body, 0)
    eb38 = wbase + 38 * CH
    wait_in(eb38, idxA, msgA, siA)
    pltpu.sync_copy(msgA, shared_v.at[idxA], add=True)

    @pl.when(wid < NMINI)
    def _():
        ebt = 32 * EPW + wid * 128
        pltpu.sync_copy(dst_hbm.at[pl.ds(ebt, 128)], idxB.at[pl.ds(0, 128)])
        pltpu.sync_copy(msg_hbm.at[pl.ds(ebt, 128)], msgB.at[pl.ds(0, 128)])
        pltpu.sync_copy(msgB.at[pl.ds(0, 128)],
                        shared_v.at[idxB.at[pl.ds(0, 128)]], add=True)

    plsc.subcore_barrier()
    pltpu.sync_copy(shared_v.at[pl.ds(sid * SLICE, SLICE)], acc_hbm.at[wid])


def _sincos(v):
    t = v * 0.6366197723675814
    qf = (t + 12582912.0) - 12582912.0
    q = qf.astype(jnp.int32)
    r = v - qf * jnp.float32(1.5707963705062866)
    r = r + qf * jnp.float32(4.371138828673793e-8)
    s2 = r * r
    sinp = r + r * s2 * (-1.6666654611e-1 + s2 * (8.3321608736e-3 + s2 * (-1.9515295891e-4)))
    cosp = 1.0 + s2 * (-0.5 + s2 * (4.166664568298827e-2 + s2 * (-1.388731625493765e-3 + s2 * 2.443315711809948e-5)))
    swap = (q & 1) != 0
    sbase = jnp.where(swap, cosp, sinp)
    cbase = jnp.where(swap, sinp, cosp)
    s = jnp.where((q & 2) != 0, -sbase, sbase)
    c = jnp.where(((q + 1) & 2) != 0, -cbase, cbase)
    return s, c


def _edge_body(c_ref, w2_ref, xj_ref, xi_ref, o_ref):
    a = xi_ref[...]
    b = xj_ref[...]
    d = b - a

    def w(k):
        return c_ref[k, 0] * (w2_ref[k, 0] - w2_ref[k + 38, 0])

    def inv(v):
        return v / (v * v + 0.1)

    ia, ib, idf, iab = inv(a), inv(b), inv(d), inv(a * b)
    sa, ca = _sincos(a)
    sb, cb = _sincos(b)
    ta, tb = jnp.tanh(a), jnp.tanh(b)
    tand = (tb - ta) / jnp.maximum(1.0 - ta * tb, 1e-20)
    tans = (ta + tb) / jnp.maximum(1.0 + ta * tb, 1e-20)
    p1, p2 = sa * cb, ca * sb
    p3, p4 = ca * cb, sa * sb
    A, B = w(17) + w(32), w(18) + w(33)
    C, D = w(20) + w(35), w(21) + w(36)
    acc = (C - A) * p1 + (A + C) * p2 + (B + D) * p3 + (B - D) * p4
    wd = w(2) + w(25)
    acc += (w(0) + w(23) - wd) * a + (w(1) + w(24) + wd) * b + w(22)
    w5 = w(5)
    ab = a * b
    acc += (w(3) + w5) * (a * a) + (w(4) + w5) * (b * b) + (w(14) - 2.0 * w5) * ab
    acc += w(15) * (ab * a) + w(16) * (ab * b)
    acc += w(6) * ia + w(7) * ib + w(8) * idf + w(9) * iab
    acc += w(10) * (ia * ia) + w(11) * (ib * ib)
    acc += w(12) * (idf * idf) + w(13) * (iab * iab)
    acc += (w(19) + w(34)) * tand + w(37) * tans
    acc += w(26) * sa + w(27) * ca + w(28) * ta
    acc += w(29) * sb + w(30) * cb + w(31) * tb
    o_ref[...] = acc


_EBR = 1000
_edge_call = pl.pallas_call(
    _edge_body,
    grid=(pl.cdiv(HROWS, _EBR),),
    in_specs=[
        pl.BlockSpec((38, 1), lambda i: (0, 0)),
        pl.BlockSpec((76, 1), lambda i: (0, 0)),
        pl.BlockSpec((_EBR, 128), lambda i: (i, 0)),
        pl.BlockSpec((_EBR, 128), lambda i: (i, 0)),
    ],
    out_specs=pl.BlockSpec((_EBR, 128), lambda i: (i, 0)),
    out_shape=jax.ShapeDtypeStruct((HROWS, 128), jnp.float32),
)


def _combine_body(f_ref, w2_ref, acc0_ref, acc1_ref, x_ref, o_ref):
    x = x_ref[...]

    def w(k):
        return f_ref[k, 0] * (w2_ref[k, 0] - w2_ref[k + 12, 0])

    iv = x / (x * x + 0.1)
    g = w(0) + (w(1) + w(3)) * x + w(2) * jnp.sign(x)
    g += w(4) * (x * x) + w(5) * (x * x * x)
    g += w(6) * iv + w(7) * (iv * iv) + w(8) * (iv * iv * iv)
    g += w(9) * jnp.sin(x) + w(10) * jnp.cos(x) + w(11) * jnp.tanh(x)
    o_ref[...] = (acc0_ref[0] + acc0_ref[1]) + (acc1_ref[0] + acc1_ref[1]) + 0.1 * g


_CBR = 200
_combine_call = pl.pallas_call(
    _combine_body,
    grid=(NPAD // 128 // _CBR,),
    in_specs=[
        pl.BlockSpec((12, 1), lambda i: (0, 0)),
        pl.BlockSpec((24, 1), lambda i: (0, 0)),
        pl.BlockSpec((2, _CBR, 128), lambda i: (0, i, 0)),
        pl.BlockSpec((2, _CBR, 128), lambda i: (0, i, 0)),
        pl.BlockSpec((_CBR, 128), lambda i: (i, 0)),
    ],
    out_specs=pl.BlockSpec((_CBR, 128), lambda i: (i, 0)),
    out_shape=jax.ShapeDtypeStruct((NPAD // 128, 128), jnp.float32),
)

_gather_h0 = _make_gather(0)
_gather_h1 = _make_gather(EH)


def kernel(t, x, edge_index, c_mask, f_mask, wc_2, wf_2):
    x_pad = jnp.pad(x.reshape(-1), (0, NPAD - N_NODES))
    xj0, xi0, dst0 = _gather_h0(x_pad, edge_index)
    xj1, xi1, dst1 = _gather_h1(x_pad, edge_index)
    msg0 = _edge_call(c_mask, wc_2, xj0.reshape(HROWS, 128), xi0.reshape(HROWS, 128))
    msg1 = _edge_call(c_mask, wc_2, xj1.reshape(HROWS, 128), xi1.reshape(HROWS, 128))
    acc0 = _scatter_call(msg0.reshape(-1), dst0)
    acc1 = _scatter_call(msg1.reshape(-1), dst1)
    out = _combine_call(f_mask, wf_2,
                        acc0.reshape(2, NPAD // 128, 128),
                        acc1.reshape(2, NPAD // 128, 128),
                        x_pad.reshape(NPAD // 128, 128))
    return out.reshape(-1)[:N_NODES].reshape(N_NODES, 1)


# speedup vs baseline: 172.9197x; 1.0349x over previous
import functools

import jax
import jax.numpy as jnp
from jax import lax
from jax.experimental import pallas as pl
from jax.experimental.pallas import tpu as pltpu
from jax.experimental.pallas import tpu_sc as plsc

N_NODES = 100000
N_EDGES = 1600000
NPAD = 102400
EH = N_EDGES // 2
HROWS = EH // 128
CH = 640
NCH = 39
EPW = NCH * CH
REM = EH - 32 * EPW
NMINI = REM // 128
SLICE = NPAD // 16

_mesh = plsc.VectorSubcoreMesh(core_axis_name="c", subcore_axis_name="s")
_sc_params = pltpu.CompilerParams(needs_layout_passes=False)


def _make_gather(e0):
    @functools.partial(
        pl.kernel,
        out_type=(
            jax.ShapeDtypeStruct((EH,), jnp.float32),
            jax.ShapeDtypeStruct((EH,), jnp.float32),
            jax.ShapeDtypeStruct((EH,), jnp.int32),
        ),
        mesh=_mesh,
        scratch_types=[
            pltpu.VMEM((NPAD,), jnp.float32),
            pltpu.VMEM((2, CH), jnp.int32),
            pltpu.VMEM((2, CH), jnp.int32),
            pltpu.VMEM((CH,), jnp.float32),
            pltpu.VMEM((CH,), jnp.float32),
            pltpu.VMEM((CH,), jnp.float32),
            pltpu.VMEM((CH,), jnp.float32),
            pltpu.VMEM((CH,), jnp.int32),
            pltpu.VMEM((CH,), jnp.int32),
            pltpu.VMEM_SHARED((NPAD,), jnp.float32),
            pltpu.SemaphoreType.DMA,
            pltpu.SemaphoreType.DMA,
            pltpu.SemaphoreType.DMA,
            pltpu.SemaphoreType.DMA,
        ],
        compiler_params=_sc_params,
    )
    def gather_kernel(x_hbm, ei_hbm, xj_hbm, xi_hbm, dst_hbm,
                      table_v, eiA, eiB, xjA, xiA, xjB, xiB, dstA, dstB,
                      table_s, siA, siB, soA, soB):
        cid = lax.axis_index("c")
        sid = lax.axis_index("s")
        wid = cid * 16 + sid
        wbase = e0 + wid * EPW

        def start_in(eb, ei_v, sem, n=CH):
            pltpu.async_copy(ei_hbm.at[:, pl.ds(eb, n)], ei_v.at[:, pl.ds(0, n)], sem)

        def wait_in(eb, ei_v, sem, n=CH):
            pltpu.make_async_copy(ei_hbm.at[:, pl.ds(eb, n)], ei_v.at[:, pl.ds(0, n)], sem).wait()

        def start_out(eb, xj_v, xi_v, dst_v, sem, n=CH):
            pltpu.async_copy(xj_v.at[pl.ds(0, n)], xj_hbm.at[pl.ds(eb - e0, n)], sem)
            pltpu.async_copy(xi_v.at[pl.ds(0, n)], xi_hbm.at[pl.ds(eb - e0, n)], sem)
            pltpu.async_copy(dst_v.at[pl.ds(0, n)], dst_hbm.at[pl.ds(eb - e0, n)], sem)

        def wait_out(eb, xj_v, xi_v, dst_v, sem, n=CH):
            pltpu.make_async_copy(xj_v.at[pl.ds(0, n)], xj_hbm.at[pl.ds(eb - e0, n)], sem).wait()
            pltpu.make_async_copy(xi_v.at[pl.ds(0, n)], xi_hbm.at[pl.ds(eb - e0, n)], sem).wait()
            pltpu.make_async_copy(dst_v.at[pl.ds(0, n)], dst_hbm.at[pl.ds(eb - e0, n)], sem).wait()

        def gather(ei_v, xj_v, xi_v, dst_v, ngrp=CH // 16):
            for g in range(ngrp):
                sl = pl.ds(g * 16, 16)
                d_idx = ei_v[1, sl]
                xj_v[sl] = plsc.load_gather(table_v, [ei_v[0, sl]])
                xi_v[sl] = plsc.load_gather(table_v, [d_idx])
                dst_v[sl] = d_idx

        start_in(wbase, eiA, siA)
        start_in(wbase + CH, eiB, siB)

        @pl.when(sid == 0)
        def _():
            pltpu.sync_copy(x_hbm, table_s)

        plsc.subcore_barrier()
        pltpu.sync_copy(table_s, table_v)

        def body(k, _):
            ebA = wbase + (2 * k) * CH
            ebB = wbase + (2 * k + 1) * CH
            wait_in(ebA, eiA, siA)

            @pl.when(k > 0)
            def _():
                wait_out(ebA, xjA, xiA, dstA, soA)

            gather(eiA, xjA, xiA, dstA)
            start_out(ebA, xjA, xiA, dstA, soA)
            start_in(ebA + 2 * CH, eiA, siA)

            wait_in(ebB, eiB, siB)

            @pl.when(k > 0)
            def _():
                wait_out(ebB, xjB, xiB, dstB, soB)

            gather(eiB, xjB, xiB, dstB)
            start_out(ebB, xjB, xiB, dstB, soB)

            @pl.when(k < 18)
            def _():
                start_in(ebB + 2 * CH, eiB, siB)

            return 0

        lax.fori_loop(0, 19, body, 0)

        eb38 = wbase + 38 * CH
        wait_in(eb38, eiA, siA)
        wait_out(eb38, xjA, xiA, dstA, soA)
        gather(eiA, xjA, xiA, dstA)
        start_out(eb38, xjA, xiA, dstA, soA)
        wait_out(wbase, xjB, xiB, dstB, soB)

        @pl.when(wid < NMINI)
        def _():
            ebt = e0 + 32 * EPW + wid * 128
            pltpu.sync_copy(ei_hbm.at[:, pl.ds(ebt, 128)], eiB.at[:, pl.ds(0, 128)])
            gather(eiB, xjB, xiB, dstB, ngrp=128 // 16)
            pltpu.sync_copy(xjB.at[pl.ds(0, 128)], xj_hbm.at[pl.ds(ebt - e0, 128)])
            pltpu.sync_copy(xiB.at[pl.ds(0, 128)], xi_hbm.at[pl.ds(ebt - e0, 128)])
            pltpu.sync_copy(dstB.at[pl.ds(0, 128)], dst_hbm.at[pl.ds(ebt - e0, 128)])

        wait_out(eb38, xjA, xiA, dstA, soA)

    return gather_kernel


@functools.partial(
    pl.kernel,
    out_type=jax.ShapeDtypeStruct((32, SLICE), jnp.float32),
    mesh=_mesh,
    scratch_types=[
        pltpu.VMEM_SHARED((NPAD,), jnp.float32),
        pltpu.VMEM((CH,), jnp.int32),
        pltpu.VMEM((CH,), jnp.int32),
        pltpu.VMEM((CH,), jnp.float32),
        pltpu.VMEM((CH,), jnp.float32),
        pltpu.VMEM((SLICE,), jnp.float32),
        pltpu.SemaphoreType.DMA,
        pltpu.SemaphoreType.DMA,
    ],
    compiler_params=_sc_params,
)
def _scatter_call(msg_hbm, dst_hbm, acc_hbm,
                  shared_v, idxA, idxB, msgA, msgB, zbuf, siA, siB):
    cid = lax.axis_index("c")
    sid = lax.axis_index("s")
    wid = cid * 16 + sid
    wbase = wid * EPW

    def start_in(eb, idx_v, msg_v, sem, n=CH):
        pltpu.async_copy(dst_hbm.at[pl.ds(eb, n)], idx_v.at[pl.ds(0, n)], sem)
        pltpu.async_copy(msg_hbm.at[pl.ds(eb, n)], msg_v.at[pl.ds(0, n)], sem)

    def wait_in(eb, idx_v, msg_v, sem, n=CH):
        pltpu.make_async_copy(dst_hbm.at[pl.ds(eb, n)], idx_v.at[pl.ds(0, n)], sem).wait()
        pltpu.make_async_copy(msg_hbm.at[pl.ds(eb, n)], msg_v.at[pl.ds(0, n)], sem).wait()

    start_in(wbase, idxA, msgA, siA)
    start_in(wbase + CH, idxB, msgB, siB)

    def zfill(i, _):
        zbuf[pl.ds(i * 16, 16)] = jnp.zeros((16,), jnp.float32)
        return 0

    lax.fori_loop(0, SLICE // 16, zfill, 0)
    pltpu.sync_copy(zbuf, shared_v.at[pl.ds(sid * SLICE, SLICE)])
    plsc.subcore_barrier()

    def body(k, _):
        ebA = wbase + (2 * k) * CH
        ebB = wbase + (2 * k + 1) * CH
        wait_in(ebA, idxA, msgA, siA)
        pltpu.sync_copy(msgA, shared_v.at[idxA], add=True)
        start_in(ebA + 2 * CH, idxA, msgA, siA)
        wait_in(ebB, idxB, msgB, siB)
        pltpu.sync_copy(msgB, shared_v.at[idxB], add=True)

        @pl.when(k < 18)
        def _():
            start_in(ebB + 2 * CH, idxB, msgB, siB)

        return 0

    lax.fori_loop(0, 19, body, 0)
    eb38 = wbase + 38 * CH
    wait_in(eb38, idxA, msgA, siA)
    pltpu.sync_copy(msgA, shared_v.at[idxA], add=True)

    @pl.when(wid < NMINI)
    def _():
        ebt = 32 * EPW + wid * 128
        pltpu.sync_copy(dst_hbm.at[pl.ds(ebt, 128)], idxB.at[pl.ds(0, 128)])
        pltpu.sync_copy(msg_hbm.at[pl.ds(ebt, 128)], msgB.at[pl.ds(0, 128)])
        pltpu.sync_copy(msgB.at[pl.ds(0, 128)],
                        shared_v.at[idxB.at[pl.ds(0, 128)]], add=True)

    plsc.subcore_barrier()
    pltpu.sync_copy(shared_v.at[pl.ds(sid * SLICE, SLICE)], acc_hbm.at[wid])


def _sincos(v):
    t = v * 0.6366197723675814
    qf = (t + 12582912.0) - 12582912.0
    q = qf.astype(jnp.int32)
    r = v - qf * jnp.float32(1.5707963705062866)
    r = r + qf * jnp.float32(4.371138828673793e-8)
    s2 = r * r
    sinp = r + r * s2 * (-1.6666654611e-1 + s2 * (8.3321608736e-3 + s2 * (-1.9515295891e-4)))
    cosp = 1.0 + s2 * (-0.5 + s2 * (4.166664568298827e-2 + s2 * (-1.388731625493765e-3 + s2 * 2.443315711809948e-5)))
    swap = (q & 1) != 0
    sbase = jnp.where(swap, cosp, sinp)
    cbase = jnp.where(swap, sinp, cosp)
    s = jnp.where((q & 2) != 0, -sbase, sbase)
    c = jnp.where(((q + 1) & 2) != 0, -cbase, cbase)
    return s, c


def _edge_body(c_ref, w2_ref, xj_ref, xi_ref, o_ref):
    a = xi_ref[...]
    b = xj_ref[...]
    d = b - a

    def w(k):
        return c_ref[k, 0] * (w2_ref[k, 0] - w2_ref[k + 38, 0])

    def inv(v):
        return v / (v * v + 0.1)

    ia, ib, idf, iab = inv(a), inv(b), inv(d), inv(a * b)
    sa, ca = _sincos(a)
    sb, cb = _sincos(b)
    ta, tb = jnp.tanh(a), jnp.tanh(b)
    tand = (tb - ta) / jnp.maximum(1.0 - ta * tb, 1e-20)
    tans = (ta + tb) / jnp.maximum(1.0 + ta * tb, 1e-20)
    p1, p2 = sa * cb, ca * sb
    p3, p4 = ca * cb, sa * sb
    A, B = w(17) + w(32), w(18) + w(33)
    C, D = w(20) + w(35), w(21) + w(36)
    acc = (C - A) * p1 + (A + C) * p2 + (B + D) * p3 + (B - D) * p4
    wd = w(2) + w(25)
    acc += (w(0) + w(23) - wd) * a + (w(1) + w(24) + wd) * b + w(22)
    w5 = w(5)
    ab = a * b
    acc += (w(3) + w5) * (a * a) + (w(4) + w5) * (b * b) + (w(14) - 2.0 * w5) * ab
    acc += w(15) * (ab * a) + w(16) * (ab * b)
    acc += w(6) * ia + w(7) * ib + w(8) * idf + w(9) * iab
    acc += w(10) * (ia * ia) + w(11) * (ib * ib)
    acc += w(12) * (idf * idf) + w(13) * (iab * iab)
    acc += (w(19) + w(34)) * tand + w(37) * tans
    acc += w(26) * sa + w(27) * ca + w(28) * ta
    acc += w(29) * sb + w(30) * cb + w(31) * tb
    o_ref[...] = acc


_EBR = 1000
_edge_call = pl.pallas_call(
    _edge_body,
    grid=(pl.cdiv(HROWS, _EBR),),
    in_specs=[
        pl.BlockSpec((38, 1), lambda i: (0, 0)),
        pl.BlockSpec((76, 1), lambda i: (0, 0)),
        pl.BlockSpec((_EBR, 128), lambda i: (i, 0)),
        pl.BlockSpec((_EBR, 128), lambda i: (i, 0)),
    ],
    out_specs=pl.BlockSpec((_EBR, 128), lambda i: (i, 0)),
    out_shape=jax.ShapeDtypeStruct((HROWS, 128), jnp.float32),
)


def _combine_body(f_ref, w2_ref, acc0_ref, acc1_ref, x_ref, o_ref):
    x = x_ref[...]

    def w(k):
        return f_ref[k, 0] * (w2_ref[k, 0] - w2_ref[k + 12, 0])

    iv = x / (x * x + 0.1)
    g = w(0) + (w(1) + w(3)) * x + w(2) * jnp.sign(x)
    g += w(4) * (x * x) + w(5) * (x * x * x)
    g += w(6) * iv + w(7) * (iv * iv) + w(8) * (iv * iv * iv)
    g += w(9) * jnp.sin(x) + w(10) * jnp.cos(x) + w(11) * jnp.tanh(x)
    o_ref[...] = (acc0_ref[0] + acc0_ref[1]) + (acc1_ref[0] + acc1_ref[1]) + 0.1 * g


_CBR = 200
_combine_call = pl.pallas_call(
    _combine_body,
    grid=(NPAD // 128 // _CBR,),
    in_specs=[
        pl.BlockSpec((12, 1), lambda i: (0, 0)),
        pl.BlockSpec((24, 1), lambda i: (0, 0)),
        pl.BlockSpec((2, _CBR, 128), lambda i: (0, i, 0)),
        pl.BlockSpec((2, _CBR, 128), lambda i: (0, i, 0)),
        pl.BlockSpec((_CBR, 128), lambda i: (i, 0)),
    ],
    out_specs=pl.BlockSpec((_CBR, 128), lambda i: (i, 0)),
    out_shape=jax.ShapeDtypeStruct((NPAD // 128, 128), jnp.float32),
)

_gather_h0 = _make_gather(0)
_gather_h1 = _make_gather(EH)


def kernel(t, x, edge_index, c_mask, f_mask, wc_2, wf_2):
    x_pad = jnp.pad(x.reshape(-1), (0, NPAD - N_NODES))
    xj0, xi0, dst0 = _gather_h0(x_pad, edge_index)
    xj1, xi1, dst1 = _gather_h1(x_pad, edge_index)
    msg0 = _edge_call(c_mask, wc_2, xj0.reshape(HROWS, 128), xi0.reshape(HROWS, 128))
    msg1 = _edge_call(c_mask, wc_2, xj1.reshape(HROWS, 128), xi1.reshape(HROWS, 128))
    acc0 = _scatter_call(msg0.reshape(-1), dst0)
    acc1 = _scatter_call(msg1.reshape(-1), dst1)
    out = _combine_call(f_mask, wf_2,
                        acc0.reshape(2, NPAD // 128, 128),
                        acc1.reshape(2, NPAD // 128, 128),
                        x_pad.reshape(NPAD // 128, 128))
    return out.reshape(-1)[:N_NODES].reshape(N_NODES, 1)


# speedup vs baseline: 183.9565x; 1.0638x over previous
import functools

import jax
import jax.numpy as jnp
from jax import lax
from jax.experimental import pallas as pl
from jax.experimental.pallas import tpu as pltpu
from jax.experimental.pallas import tpu_sc as plsc

N_NODES = 100000
N_EDGES = 1600000
NPAD = 102400
EH = N_EDGES // 2
HROWS = EH // 128
CH = 640
NCH = 39
EPW = NCH * CH
REM = EH - 32 * EPW
NMINI = REM // 128
SLICE = NPAD // 16

_mesh = plsc.VectorSubcoreMesh(core_axis_name="c", subcore_axis_name="s")
_sc_params = pltpu.CompilerParams(needs_layout_passes=False)


def _make_gather(e0):
    @functools.partial(
        pl.kernel,
        out_type=(
            jax.ShapeDtypeStruct((EH,), jnp.float32),
            jax.ShapeDtypeStruct((EH,), jnp.float32),
            jax.ShapeDtypeStruct((EH,), jnp.int32),
        ),
        mesh=_mesh,
        scratch_types=[
            pltpu.VMEM((NPAD,), jnp.float32),
            pltpu.VMEM((2, CH), jnp.int32),
            pltpu.VMEM((2, CH), jnp.int32),
            pltpu.VMEM((CH,), jnp.float32),
            pltpu.VMEM((CH,), jnp.float32),
            pltpu.VMEM((CH,), jnp.float32),
            pltpu.VMEM((CH,), jnp.float32),
            pltpu.VMEM((CH,), jnp.int32),
            pltpu.VMEM((CH,), jnp.int32),
            pltpu.VMEM_SHARED((NPAD,), jnp.float32),
            pltpu.SemaphoreType.DMA,
            pltpu.SemaphoreType.DMA,
            pltpu.SemaphoreType.DMA,
            pltpu.SemaphoreType.DMA,
        ],
        compiler_params=_sc_params,
    )
    def gather_kernel(x_hbm, ei_hbm, xj_hbm, xi_hbm, dst_hbm,
                      table_v, eiA, eiB, xjA, xiA, xjB, xiB, dstA, dstB,
                      table_s, siA, siB, soA, soB):
        cid = lax.axis_index("c")
        sid = lax.axis_index("s")
        wid = cid * 16 + sid
        wbase = e0 + wid * EPW

        def start_in(eb, ei_v, sem, n=CH):
            pltpu.async_copy(ei_hbm.at[:, pl.ds(eb, n)], ei_v.at[:, pl.ds(0, n)], sem)

        def wait_in(eb, ei_v, sem, n=CH):
            pltpu.make_async_copy(ei_hbm.at[:, pl.ds(eb, n)], ei_v.at[:, pl.ds(0, n)], sem).wait()

        def start_out(eb, xj_v, xi_v, dst_v, sem, n=CH):
            pltpu.async_copy(xj_v.at[pl.ds(0, n)], xj_hbm.at[pl.ds(eb - e0, n)], sem)
            pltpu.async_copy(xi_v.at[pl.ds(0, n)], xi_hbm.at[pl.ds(eb - e0, n)], sem)
            pltpu.async_copy(dst_v.at[pl.ds(0, n)], dst_hbm.at[pl.ds(eb - e0, n)], sem)

        def wait_out(eb, xj_v, xi_v, dst_v, sem, n=CH):
            pltpu.make_async_copy(xj_v.at[pl.ds(0, n)], xj_hbm.at[pl.ds(eb - e0, n)], sem).wait()
            pltpu.make_async_copy(xi_v.at[pl.ds(0, n)], xi_hbm.at[pl.ds(eb - e0, n)], sem).wait()
            pltpu.make_async_copy(dst_v.at[pl.ds(0, n)], dst_hbm.at[pl.ds(eb - e0, n)], sem).wait()

        def gather(ei_v, xj_v, xi_v, dst_v, ngrp=CH // 16):
            for g in range(ngrp):
                sl = pl.ds(g * 16, 16)
                d_idx = ei_v[1, sl]
                xj_v[sl] = plsc.load_gather(table_v, [ei_v[0, sl]])
                xi_v[sl] = plsc.load_gather(table_v, [d_idx])
                dst_v[sl] = d_idx

        start_in(wbase, eiA, siA)
        start_in(wbase + CH, eiB, siB)

        @pl.when(sid == 0)
        def _():
            pltpu.sync_copy(x_hbm, table_s)

        plsc.subcore_barrier()
        pltpu.sync_copy(table_s, table_v)

        def body(k, _):
            ebA = wbase + (2 * k) * CH
            ebB = wbase + (2 * k + 1) * CH
            wait_in(ebA, eiA, siA)

            @pl.when(k > 0)
            def _():
                wait_out(ebA, xjA, xiA, dstA, soA)

            gather(eiA, xjA, xiA, dstA)
            start_out(ebA, xjA, xiA, dstA, soA)
            start_in(ebA + 2 * CH, eiA, siA)

            wait_in(ebB, eiB, siB)

            @pl.when(k > 0)
            def _():
                wait_out(ebB, xjB, xiB, dstB, soB)

            gather(eiB, xjB, xiB, dstB)
            start_out(ebB, xjB, xiB, dstB, soB)

            @pl.when(k < 18)
            def _():
                start_in(ebB + 2 * CH, eiB, siB)

            return 0

        lax.fori_loop(0, 19, body, 0)

        eb38 = wbase + 38 * CH
        wait_in(eb38, eiA, siA)
        wait_out(eb38, xjA, xiA, dstA, soA)
        gather(eiA, xjA, xiA, dstA)
        start_out(eb38, xjA, xiA, dstA, soA)
        wait_out(wbase, xjB, xiB, dstB, soB)

        @pl.when(wid < NMINI)
        def _():
            ebt = e0 + 32 * EPW + wid * 128
            pltpu.sync_copy(ei_hbm.at[:, pl.ds(ebt, 128)], eiB.at[:, pl.ds(0, 128)])
            gather(eiB, xjB, xiB, dstB, ngrp=128 // 16)
            pltpu.sync_copy(xjB.at[pl.ds(0, 128)], xj_hbm.at[pl.ds(ebt - e0, 128)])
            pltpu.sync_copy(xiB.at[pl.ds(0, 128)], xi_hbm.at[pl.ds(ebt - e0, 128)])
            pltpu.sync_copy(dstB.at[pl.ds(0, 128)], dst_hbm.at[pl.ds(ebt - e0, 128)])

        wait_out(eb38, xjA, xiA, dstA, soA)

    return gather_kernel


@functools.partial(
    pl.kernel,
    out_type=jax.ShapeDtypeStruct((32, SLICE), jnp.float32),
    mesh=_mesh,
    scratch_types=[
        pltpu.VMEM_SHARED((NPAD,), jnp.float32),
        pltpu.VMEM((CH,), jnp.int32),
        pltpu.VMEM((CH,), jnp.int32),
        pltpu.VMEM((CH,), jnp.float32),
        pltpu.VMEM((CH,), jnp.float32),
        pltpu.VMEM((SLICE,), jnp.float32),
        pltpu.SemaphoreType.DMA,
        pltpu.SemaphoreType.DMA,
    ],
    compiler_params=_sc_params,
)
def _scatter_call(msg_hbm, dst_hbm, acc_hbm,
                  shared_v, idxA, idxB, msgA, msgB, zbuf, siA, siB):
    cid = lax.axis_index("c")
    sid = lax.axis_index("s")
    wid = cid * 16 + sid
    wbase = wid * EPW

    def start_in(eb, idx_v, msg_v, sem, n=CH):
        pltpu.async_copy(dst_hbm.at[pl.ds(eb, n)], idx_v.at[pl.ds(0, n)], sem)
        pltpu.async_copy(msg_hbm.at[pl.ds(eb, n)], msg_v.at[pl.ds(0, n)], sem)

    def wait_in(eb, idx_v, msg_v, sem, n=CH):
        pltpu.make_async_copy(dst_hbm.at[pl.ds(eb, n)], idx_v.at[pl.ds(0, n)], sem).wait()
        pltpu.make_async_copy(msg_hbm.at[pl.ds(eb, n)], msg_v.at[pl.ds(0, n)], sem).wait()

    start_in(wbase, idxA, msgA, siA)
    start_in(wbase + CH, idxB, msgB, siB)

    def zfill(i, _):
        zbuf[pl.ds(i * 16, 16)] = jnp.zeros((16,), jnp.float32)
        return 0

    lax.fori_loop(0, SLICE // 16, zfill, 0)
    pltpu.sync_copy(zbuf, shared_v.at[pl.ds(sid * SLICE, SLICE)])
    plsc.subcore_barrier()

    def body(k, _):
        ebA = wbase + (2 * k) * CH
        ebB = wbase + (2 * k + 1) * CH
        wait_in(ebA, idxA, msgA, siA)
        pltpu.sync_copy(msgA, shared_v.at[idxA], add=True)
        start_in(ebA + 2 * CH, idxA, msgA, siA)
        wait_in(ebB, idxB, msgB, siB)
        pltpu.sync_copy(msgB, shared_v.at[idxB], add=True)

        @pl.when(k < 18)
        def _():
            start_in(ebB + 2 * CH, idxB, msgB, siB)

        return 0

    lax.fori_loop(0, 19, body, 0)
    eb38 = wbase + 38 * CH
    wait_in(eb38, idxA, msgA, siA)
    pltpu.sync_copy(msgA, shared_v.at[idxA], add=True)

    @pl.when(wid < NMINI)
    def _():
        ebt = 32 * EPW + wid * 128
        pltpu.sync_copy(dst_hbm.at[pl.ds(ebt, 128)], idxB.at[pl.ds(0, 128)])
        pltpu.sync_copy(msg_hbm.at[pl.ds(ebt, 128)], msgB.at[pl.ds(0, 128)])
        pltpu.sync_copy(msgB.at[pl.ds(0, 128)],
                        shared_v.at[idxB.at[pl.ds(0, 128)]], add=True)

    plsc.subcore_barrier()
    pltpu.sync_copy(shared_v.at[pl.ds(sid * SLICE, SLICE)], acc_hbm.at[wid])


def _sincos(v):
    t = v * 0.6366197723675814
    qf = (t + 12582912.0) - 12582912.0
    q = qf.astype(jnp.int32)
    r = v - qf * jnp.float32(1.5707963705062866)
    r = r + qf * jnp.float32(4.371138828673793e-8)
    s2 = r * r
    sinp = r + r * s2 * (-1.6666654611e-1 + s2 * (8.3321608736e-3 + s2 * (-1.9515295891e-4)))
    cosp = 1.0 + s2 * (-0.5 + s2 * (4.166664568298827e-2 + s2 * (-1.388731625493765e-3 + s2 * 2.443315711809948e-5)))
    swap = (q & 1) != 0
    sbase = jnp.where(swap, cosp, sinp)
    cbase = jnp.where(swap, sinp, cosp)
    s = jnp.where((q & 2) != 0, -sbase, sbase)
    c = jnp.where(((q + 1) & 2) != 0, -cbase, cbase)
    return s, c


def _edge_body(c_ref, w2_ref, xj_ref, xi_ref, o_ref):
    a = xi_ref[...]
    b = xj_ref[...]
    d = b - a

    def w(k):
        return c_ref[k, 0] * (w2_ref[k, 0] - w2_ref[k + 38, 0])

    def inv(v):
        return v / (v * v + 0.1)

    ia, ib, idf, iab = inv(a), inv(b), inv(d), inv(a * b)
    sa, ca = _sincos(a)
    sb, cb = _sincos(b)
    ta, tb = jnp.tanh(a), jnp.tanh(b)
    tand = (tb - ta) / jnp.maximum(1.0 - ta * tb, 1e-20)
    tans = (ta + tb) / jnp.maximum(1.0 + ta * tb, 1e-20)
    p1, p2 = sa * cb, ca * sb
    p3, p4 = ca * cb, sa * sb
    A, B = w(17) + w(32), w(18) + w(33)
    C, D = w(20) + w(35), w(21) + w(36)
    acc = (C - A) * p1 + (A + C) * p2 + (B + D) * p3 + (B - D) * p4
    wd = w(2) + w(25)
    acc += (w(0) + w(23) - wd) * a + (w(1) + w(24) + wd) * b + w(22)
    w5 = w(5)
    ab = a * b
    acc += (w(3) + w5) * (a * a) + (w(4) + w5) * (b * b) + (w(14) - 2.0 * w5) * ab
    acc += w(15) * (ab * a) + w(16) * (ab * b)
    acc += w(6) * ia + w(7) * ib + w(8) * idf + w(9) * iab
    acc += w(10) * (ia * ia) + w(11) * (ib * ib)
    acc += w(12) * (idf * idf) + w(13) * (iab * iab)
    acc += (w(19) + w(34)) * tand + w(37) * tans
    acc += w(26) * sa + w(27) * ca + w(28) * ta
    acc += w(29) * sb + w(30) * cb + w(31) * tb
    o_ref[...] = acc


_EBR = 800
_edge_call = pl.pallas_call(
    _edge_body,
    grid=(pl.cdiv(HROWS, _EBR),),
    in_specs=[
        pl.BlockSpec((38, 1), lambda i: (0, 0)),
        pl.BlockSpec((76, 1), lambda i: (0, 0)),
        pl.BlockSpec((_EBR, 128), lambda i: (i, 0)),
        pl.BlockSpec((_EBR, 128), lambda i: (i, 0)),
    ],
    out_specs=pl.BlockSpec((_EBR, 128), lambda i: (i, 0)),
    out_shape=jax.ShapeDtypeStruct((HROWS, 128), jnp.float32),
)


def _combine_body(f_ref, w2_ref, acc0_ref, acc1_ref, x_ref, o_ref):
    x = x_ref[...]

    def w(k):
        return f_ref[k, 0] * (w2_ref[k, 0] - w2_ref[k + 12, 0])

    iv = x / (x * x + 0.1)
    g = w(0) + (w(1) + w(3)) * x + w(2) * jnp.sign(x)
    g += w(4) * (x * x) + w(5) * (x * x * x)
    g += w(6) * iv + w(7) * (iv * iv) + w(8) * (iv * iv * iv)
    g += w(9) * jnp.sin(x) + w(10) * jnp.cos(x) + w(11) * jnp.tanh(x)
    o_ref[...] = (acc0_ref[0] + acc0_ref[1]) + (acc1_ref[0] + acc1_ref[1]) + 0.1 * g


_CBR = 200
_combine_call = pl.pallas_call(
    _combine_body,
    grid=(NPAD // 128 // _CBR,),
    in_specs=[
        pl.BlockSpec((12, 1), lambda i: (0, 0)),
        pl.BlockSpec((24, 1), lambda i: (0, 0)),
        pl.BlockSpec((2, _CBR, 128), lambda i: (0, i, 0)),
        pl.BlockSpec((2, _CBR, 128), lambda i: (0, i, 0)),
        pl.BlockSpec((_CBR, 128), lambda i: (i, 0)),
    ],
    out_specs=pl.BlockSpec((_CBR, 128), lambda i: (i, 0)),
    out_shape=jax.ShapeDtypeStruct((NPAD // 128, 128), jnp.float32),
)

_gather_h0 = _make_gather(0)
_gather_h1 = _make_gather(EH)


def kernel(t, x, edge_index, c_mask, f_mask, wc_2, wf_2):
    x_pad = jnp.pad(x.reshape(-1), (0, NPAD - N_NODES))
    xj0, xi0, dst0 = _gather_h0(x_pad, edge_index)
    xj1, xi1, dst1 = _gather_h1(x_pad, edge_index)
    msg0 = _edge_call(c_mask, wc_2, xj0.reshape(HROWS, 128), xi0.reshape(HROWS, 128))
    msg1 = _edge_call(c_mask, wc_2, xj1.reshape(HROWS, 128), xi1.reshape(HROWS, 128))
    acc0 = _scatter_call(msg0.reshape(-1), dst0)
    acc1 = _scatter_call(msg1.reshape(-1), dst1)
    out = _combine_call(f_mask, wf_2,
                        acc0.reshape(2, NPAD // 128, 128),
                        acc1.reshape(2, NPAD // 128, 128),
                        x_pad.reshape(NPAD // 128, 128))
    return out.reshape(-1)[:N_NODES].reshape(N_NODES, 1)
